# baseline (speedup 1.0000x reference)
import functools

import jax
import jax.numpy as jnp
from jax import lax
from jax.experimental import pallas as pl
from jax.experimental.pallas import tpu as pltpu
from jax.experimental.pallas import tpu_sc as plsc

B = 4
LEVEL_SHAPES = [(64, 64), (32, 32), (16, 16)]

NC = 2
NSUB = 16
NW = NC * NSUB
LANES = 16
CW = 128


def _zero_buf(buf, nrows, ncols):
    def zrow(i, _):
        for j in range(ncols // LANES):
            buf[i, pl.ds(j * LANES, LANES)] = jnp.zeros((LANES,), jnp.float32)
        return 0
    lax.fori_loop(0, nrows, zrow, 0)


def _make_pad(n, c, h, w):
    hw = h * w
    RA = B * hw // 2
    bk = 5000
    assert n % bk == 0
    nb = n // bk
    sh = (h // 2).bit_length() - 1

    def body(f_ref, b_ref, y_ref, x_ref, fp_ref, ix_ref):
        f = f_ref[...]
        yv = y_ref[0]
        yh = lax.bitwise_and(yv, (h // 2) - 1)
        half = lax.shift_right_logical(yv, sh)
        r2 = b_ref[0] * (hw // 2) + yh * w + x_ref[0]
        r2 = jnp.minimum(r2, RA - 1)
        ix_ref[0] = r2
        top = half == 0
        fp_ref[:, :c] = jnp.where(top, f, 0.0)
        fp_ref[:, c:] = jnp.where(top, 0.0, f)

    return pl.pallas_call(
        body,
        grid=(nb,),
        in_specs=[
            pl.BlockSpec((bk, c), lambda i: (i, 0)),
            pl.BlockSpec((1, bk, 1), lambda i: (i, 0, 0)),
            pl.BlockSpec((1, bk, 1), lambda i: (i, 0, 0)),
            pl.BlockSpec((1, bk, 1), lambda i: (i, 0, 0)),
        ],
        out_specs=[
            pl.BlockSpec((bk, CW), lambda i: (i, 0)),
            pl.BlockSpec((1, bk, 1), lambda i: (i, 0, 0)),
        ],
        out_shape=[
            jax.ShapeDtypeStruct((n, CW), jnp.float32),
            jax.ShapeDtypeStruct((nb, bk, 1), jnp.int32),
        ],
    )


def _make_scatter(n, c, h, w, pre=False):
    hw = h * w
    R = B * hw
    pair = c < CW
    G = 1 if pair else c // CW
    assert (c == CW // 2) if pair else (c % CW == 0)
    phases = 2 if G > 2 else 1
    gpp = G // phases
    GP = 2 if pair else gpp
    RA = R // 2 if (pair or pre) else gpp * R
    K = 128 if c == CW else 64
    nblk = -(-n // K)
    tail = n - (nblk - 1) * K
    assert tail % LANES == 0
    base, ext = divmod(nblk, NW)
    last_owner = NW - 1 if base > 0 else ext - 1
    rows_per_sub = RA // NSUB
    assert RA % NSUB == 0
    CZ = min(rows_per_sub, K)

    mesh = plsc.VectorSubcoreMesh(core_axis_name="c", subcore_axis_name="s",
                                  num_cores=NC, num_subcores=NSUB)

    scratch = {
        "acc": pltpu.VMEM_SHARED((RA, CW), jnp.float32),
        "idxbuf": pltpu.VMEM((2, GP, K), jnp.int32),
        "sem0": pltpu.SemaphoreType.DMA,
        "sem1": pltpu.SemaphoreType.DMA,
        "ssem0": pltpu.SemaphoreType.DMA,
        "ssem1": pltpu.SemaphoreType.DMA,
    }
    if not pre:
        scratch["bbuf"] = pltpu.VMEM((2, K), jnp.int32)
        scratch["ybuf"] = pltpu.VMEM((2, K), jnp.int32)
        scratch["xbuf"] = pltpu.VMEM((2, K), jnp.int32)
    if pair:
        scratch["stage"] = pltpu.VMEM((2, K, c), jnp.float32)
        scratch["fbufL"] = pltpu.VMEM((K, CW), jnp.float32)
        scratch["fbufR"] = pltpu.VMEM((K, CW), jnp.float32)
    else:
        scratch["fbuf"] = pltpu.VMEM((2, gpp, K, CW), jnp.float32)
        scratch["zbuf"] = pltpu.VMEM((CZ, CW), jnp.float32)

    @functools.partial(
        pl.kernel,
        out_type=jax.ShapeDtypeStruct((NC, phases * RA, CW), jnp.float32),
        mesh=mesh,
        scratch_types=scratch,
    )
    def sc_kernel(feat, bidx, yidx, xidx, tok, out, *, acc, idxbuf,
                  sem0, sem1, ssem0, ssem1, bbuf=None, ybuf=None,
                  xbuf=None, zbuf=None, stage=None, fbufL=None,
                  fbufR=None, fbuf=None):
        del tok
        cid = lax.axis_index("c")
        sid = lax.axis_index("s")
        wid = cid * NSUB + sid
        sems = (sem0, sem1)

        if pair:
            _zero_buf(fbufL, K, CW)
            _zero_buf(fbufR, K, CW)
            zsrc = fbufL
        else:
            _zero_buf(zbuf, CZ, CW)
            zsrc = zbuf
        for t in range(rows_per_sub // CZ):
            r0 = sid * rows_per_sub + t * CZ
            pltpu.sync_copy(zsrc.at[pl.ds(0, CZ), :],
                            acc.at[pl.ds(r0, CZ), :])

        my_base = wid * base + jnp.minimum(wid, ext)
        my_cnt = base + jnp.where(wid < ext, 1, 0)
        full_cnt = my_cnt - jnp.where(wid == last_owner, 1, 0)

        def copies(bi, d, ph):
            off = (my_base + bi) * K
            sem = sems[d]
            cps = []
            if pair:
                cps.append((feat.at[pl.ds(off, K), :], stage.at[d], sem))
            else:
                for g in range(gpp):
                    gc = (ph * gpp + g) * CW
                    cps.append((feat.at[pl.ds(off, K), pl.ds(gc, CW)],
                                fbuf.at[d, g], sem))
            if pre:
                cps.append((bidx.at[pl.ds(off, K)], idxbuf.at[d, 0], sem))
            else:
                cps.append((bidx.at[pl.ds(off, K)], bbuf.at[d], sem))
                cps.append((yidx.at[pl.ds(off, K)], ybuf.at[d], sem))
                cps.append((xidx.at[pl.ds(off, K)], xbuf.at[d], sem))
            return cps

        def issue(bi, d, ph):
            @pl.when(bi < full_cnt)
            def _():
                for s, t, m in copies(bi, d, ph):
                    pltpu.async_copy(s, t, m)

        def compute_idx(krows, slot_b, slot_y, slot_x, slot_i):
            for j in range(K // LANES):
                sl = pl.ds(j * LANES, LANES)
                if j * LANES < krows:
                    if pair:
                        yv = slot_y[sl]
                        yh = lax.bitwise_and(yv, (h // 2) - 1)
                        half = lax.shift_right_logical(
                            yv, (h // 2).bit_length() - 1)
                        r2 = slot_b[sl] * (hw // 2) + yh * w + slot_x[sl]
                        r2 = jnp.minimum(r2, RA - 1)
                        slot_i[0, sl] = jnp.where(half == 0, r2, -1)
                        slot_i[1, sl] = jnp.where(half == 1, r2, -1)
                    else:
                        flat = slot_b[sl] * hw + slot_y[sl] * w + slot_x[sl]
                        flat = jnp.minimum(flat, R - 1)
                        for g in range(gpp):
                            slot_i[g, sl] = flat + g * R
                else:
                    for g in range(GP):
                        slot_i[g, sl] = jnp.full((LANES,), -1, jnp.int32)

        def scatter(d, krows):
            if not pre:
                compute_idx(krows, bbuf.at[d], ybuf.at[d], xbuf.at[d],
                            idxbuf.at[d])
            if pair:
                def shuf(i, _):
                    for j in range(c // LANES):
                        v = stage[d, i, pl.ds(j * LANES, LANES)]
                        fbufL[i, pl.ds(j * LANES, LANES)] = v
                        fbufR[i, pl.ds(c + j * LANES, LANES)] = v
                    return 0
                lax.fori_loop(0, krows, shuf, 0)
                pltpu.async_copy(
                    fbufL, acc.at[plsc.Indices(idxbuf.at[d, 0],
                                               ignored_value=-1)],
                    ssem0, add=True)
                pltpu.async_copy(
                    fbufR, acc.at[plsc.Indices(idxbuf.at[d, 1],
                                               ignored_value=-1)],
                    ssem1, add=True)
                pltpu.make_async_copy(
                    fbufL, acc.at[plsc.Indices(idxbuf.at[d, 0],
                                               ignored_value=-1)],
                    ssem0).wait()
                pltpu.make_async_copy(
                    fbufR, acc.at[plsc.Indices(idxbuf.at[d, 1],
                                               ignored_value=-1)],
                    ssem1).wait()
            else:
                ssems = (ssem0, ssem1)
                for g in range(gpp):
                    pltpu.async_copy(
                        fbuf.at[d, g],
                        acc.at[plsc.Indices(idxbuf.at[d, g],
                                            ignored_value=-1)],
                        ssems[g % 2], add=True)
                for g in range(gpp):
                    pltpu.make_async_copy(
                        fbuf.at[d, g],
                        acc.at[plsc.Indices(idxbuf.at[d, g],
                                            ignored_value=-1)],
                        ssems[g % 2]).wait()

        for ph in range(phases):
            if ph > 0:
                for t in range(rows_per_sub // CZ):
                    r0 = sid * rows_per_sub + t * CZ
                    pltpu.sync_copy(zbuf, acc.at[pl.ds(r0, CZ), :])
            plsc.subcore_barrier()

            def process(bi, d, ph=ph):
                @pl.when(bi < full_cnt)
                def _():
                    for s, t, m in copies(bi, d, ph):
                        pltpu.make_async_copy(s, t, m).wait()
                    issue(bi + 1, d ^ 1, ph)
                    scatter(d, K)

            issue(0, 0, ph)

            def pbody(p, _):
                process(2 * p, 0)
                process(2 * p + 1, 1)
                return 0
            lax.fori_loop(0, (full_cnt + 1) // 2, pbody, 0)

            @pl.when(wid == last_owner)
            def _():
                off = (nblk - 1) * K
                if pair:
                    pltpu.sync_copy(feat.at[pl.ds(off, tail), :],
                                    stage.at[0, pl.ds(0, tail), :])
                else:
                    for g in range(gpp):
                        gc = (ph * gpp + g) * CW
                        pltpu.sync_copy(feat.at[pl.ds(off, tail),
                                                pl.ds(gc, CW)],
                                        fbuf.at[0, g, pl.ds(0, tail), :])
                if pre:
                    assert tail == K
                    pltpu.sync_copy(bidx.at[pl.ds(off, tail)],
                                    idxbuf.at[0, 0])
                else:
                    pltpu.sync_copy(bidx.at[pl.ds(off, tail)],
                                    bbuf.at[0, pl.ds(0, tail)])
                    pltpu.sync_copy(yidx.at[pl.ds(off, tail)],
                                    ybuf.at[0, pl.ds(0, tail)])
                    pltpu.sync_copy(xidx.at[pl.ds(off, tail)],
                                    xbuf.at[0, pl.ds(0, tail)])
                scatter(0, tail)

            plsc.subcore_barrier()

            for t in range(rows_per_sub // CZ):
                r0 = sid * rows_per_sub + t * CZ
                pltpu.sync_copy(acc.at[pl.ds(r0, CZ), :],
                                out.at[cid, pl.ds(ph * RA + r0, CZ), :])

    return sc_kernel


def _make_adapter(hw, cin, cout):
    pair = cin < CW
    G = 1 if pair else cin // CW

    if pair:
        def body(p_ref, w_ref, b_ref, o_ref):
            d = p_ref[0, 0] + p_ref[1, 0]
            we = w_ref[...]
            ot = lax.dot_general(we, d[:, :cin], (((1,), (1,)), ((), ())),
                                 preferred_element_type=jnp.float32)
            ob = lax.dot_general(we, d[:, cin:], (((1,), (1,)), ((), ())),
                                 preferred_element_type=jnp.float32)
            bb = b_ref[0][:, None]
            o_ref[0, :, :hw // 2] = ot + bb
            o_ref[0, :, hw // 2:] = ob + bb

        return pl.pallas_call(
            body,
            grid=(B,),
            in_specs=[
                pl.BlockSpec((NC, 1, hw // 2, CW), lambda b: (0, b, 0, 0)),
                pl.BlockSpec((cout, cin), lambda b: (0, 0)),
                pl.BlockSpec((1, cout), lambda b: (0, 0)),
            ],
            out_specs=pl.BlockSpec((1, cout, hw), lambda b: (b, 0, 0)),
            out_shape=jax.ShapeDtypeStruct((B, cout, hw), jnp.float32),
        )

    def body(p_ref, w_ref, b_ref, o_ref):
        o = None
        for g in range(G):
            d = p_ref[0, g, 0] + p_ref[1, g, 0]
            wg = w_ref[:, g * CW:(g + 1) * CW]
            part = lax.dot_general(wg, d, (((1,), (1,)), ((), ())),
                                   preferred_element_type=jnp.float32)
            o = part if o is None else o + part
        o_ref[0] = o + b_ref[0][:, None]

    return pl.pallas_call(
        body,
        grid=(B,),
        in_specs=[
            pl.BlockSpec((NC, G, 1, hw, CW), lambda b: (0, 0, b, 0, 0)),
            pl.BlockSpec((cout, cin), lambda b: (0, 0)),
            pl.BlockSpec((1, cout), lambda b: (0, 0)),
        ],
        out_specs=pl.BlockSpec((1, cout, hw), lambda b: (b, 0, 0)),
        out_shape=jax.ShapeDtypeStruct((B, cout, hw), jnp.float32),
    )


def _level(feat, bidx, yidx, xidx, W, bvec, h, w, tok):
    n, c = feat.shape
    cout = W.shape[0]
    hw = h * w
    pair = c < CW
    G = 1 if pair else c // CW
    bidx = bidx.astype(jnp.int32)
    yidx = yidx.astype(jnp.int32)
    xidx = xidx.astype(jnp.int32)
    if pair:
        padk = _make_pad(n, c, h, w)
        bk = 5000
        nb = n // bk
        fpad, pidx = padk(feat,
                          bidx.reshape(nb, bk, 1),
                          yidx.reshape(nb, bk, 1),
                          xidx.reshape(nb, bk, 1))
        sc = _make_scatter(n, CW, h, w, pre=True)
        parts = sc(fpad, pidx.reshape(-1), yidx, xidx, tok)
        parts = parts.reshape(NC, B, hw // 2, CW)
    else:
        sc = _make_scatter(n, c, h, w)
        parts = sc(feat, bidx, yidx, xidx, tok)
        parts = parts.reshape(NC, G, B, hw, CW)
    mm = _make_adapter(hw, c, cout)
    out = mm(parts, W, bvec.reshape(1, cout))
    return out.reshape(B, cout, h, w), lax.slice(parts.reshape(-1), (0,), (8,))


def kernel(feat0, bidx0, yidx0, xidx0, feat1, bidx1, yidx1, xidx1,
           feat2, bidx2, yidx2, xidx2, W0, b0, W1, b1, W2, b2, batch_size):
    del batch_size
    tok = jnp.zeros((8,), jnp.float32)
    out1, tok = _level(feat1, bidx1, yidx1, xidx1, W1, b1,
                       *LEVEL_SHAPES[1], tok)
    out2, tok = _level(feat2, bidx2, yidx2, xidx2, W2, b2,
                       *LEVEL_SHAPES[2], tok)
    out0, _ = _level(feat0, bidx0, yidx0, xidx0, W0, b0,
                     *LEVEL_SHAPES[0], tok)
    return (out0, out1, out2)


# speedup vs baseline: 3.5949x; 3.5949x over previous
import functools

import jax
import jax.numpy as jnp
from jax import lax
from jax.experimental import pallas as pl
from jax.experimental.pallas import tpu as pltpu
from jax.experimental.pallas import tpu_sc as plsc

B = 4
LEVEL_SHAPES = [(64, 64), (32, 32), (16, 16)]

NC = 2
NSUB = 16
NW = NC * NSUB
LANES = 16
CW = 128


def _zero_buf(buf, nrows, ncols):
    def zrow(i, _):
        for j in range(ncols // LANES):
            buf[i, pl.ds(j * LANES, LANES)] = jnp.zeros((LANES,), jnp.float32)
        return 0
    lax.fori_loop(0, nrows, zrow, 0)


def _make_scatter(n, c, h, w, pre=False):
    hw = h * w
    R = B * hw
    pair = c < CW
    G = 1 if pair else c // CW
    assert (c == CW // 2) if pair else (c % CW == 0)
    phases = 2 if G > 2 else 1
    gpp = G // phases
    GP = 2 if pair else gpp
    RA = R // 2 if (pair or pre) else gpp * R
    K = 128 if c == CW else 64
    nblk = -(-n // K)
    tail = n - (nblk - 1) * K
    assert tail % LANES == 0
    base, ext = divmod(nblk, NW)
    last_owner = NW - 1 if base > 0 else ext - 1
    rows_per_sub = RA // NSUB
    assert RA % NSUB == 0
    CZ = min(rows_per_sub, K)

    mesh = plsc.VectorSubcoreMesh(core_axis_name="c", subcore_axis_name="s",
                                  num_cores=NC, num_subcores=NSUB)

    scratch = {
        "acc": pltpu.VMEM_SHARED((RA, CW), jnp.float32),
        "idxbuf": pltpu.VMEM((2, GP, K), jnp.int32),
        "sem0": pltpu.SemaphoreType.DMA,
        "sem1": pltpu.SemaphoreType.DMA,
        "ssem0": pltpu.SemaphoreType.DMA,
        "ssem1": pltpu.SemaphoreType.DMA,
    }
    if not pre:
        scratch["bbuf"] = pltpu.VMEM((2, K), jnp.int32)
        scratch["ybuf"] = pltpu.VMEM((2, K), jnp.int32)
        scratch["xbuf"] = pltpu.VMEM((2, K), jnp.int32)
    if pair:
        scratch["stage"] = pltpu.VMEM((2, K, c), jnp.float32)
        scratch["fbufL"] = pltpu.VMEM((K, CW), jnp.float32)
        scratch["fbufR"] = pltpu.VMEM((K, CW), jnp.float32)
    else:
        scratch["fbuf"] = pltpu.VMEM((2, gpp, K, CW), jnp.float32)
        scratch["zbuf"] = pltpu.VMEM((CZ, CW), jnp.float32)

    @functools.partial(
        pl.kernel,
        out_type=jax.ShapeDtypeStruct((NC, phases * RA, CW), jnp.float32),
        mesh=mesh,
        scratch_types=scratch,
    )
    def sc_kernel(feat, bidx, yidx, xidx, tok, out, *, acc, idxbuf,
                  sem0, sem1, ssem0, ssem1, bbuf=None, ybuf=None,
                  xbuf=None, zbuf=None, stage=None, fbufL=None,
                  fbufR=None, fbuf=None):
        del tok
        cid = lax.axis_index("c")
        sid = lax.axis_index("s")
        wid = cid * NSUB + sid
        sems = (sem0, sem1)

        if pair:
            _zero_buf(fbufL, K, CW)
            _zero_buf(fbufR, K, CW)
            zsrc = fbufL
        else:
            _zero_buf(zbuf, CZ, CW)
            zsrc = zbuf
        for t in range(rows_per_sub // CZ):
            r0 = sid * rows_per_sub + t * CZ
            pltpu.sync_copy(zsrc.at[pl.ds(0, CZ), :],
                            acc.at[pl.ds(r0, CZ), :])

        my_base = wid * base + jnp.minimum(wid, ext)
        my_cnt = base + jnp.where(wid < ext, 1, 0)
        full_cnt = my_cnt - jnp.where(wid == last_owner, 1, 0)

        def copies(bi, d, ph):
            off = (my_base + bi) * K
            sem = sems[d]
            cps = []
            if pair:
                cps.append((feat.at[pl.ds(off, K), :], stage.at[d], sem))
            else:
                for g in range(gpp):
                    gc = (ph * gpp + g) * CW
                    cps.append((feat.at[pl.ds(off, K), pl.ds(gc, CW)],
                                fbuf.at[d, g], sem))
            if pre:
                cps.append((bidx.at[pl.ds(off, K)], idxbuf.at[d, 0], sem))
            else:
                cps.append((bidx.at[pl.ds(off, K)], bbuf.at[d], sem))
                cps.append((yidx.at[pl.ds(off, K)], ybuf.at[d], sem))
                cps.append((xidx.at[pl.ds(off, K)], xbuf.at[d], sem))
            return cps

        def issue(bi, d, ph):
            @pl.when(bi < full_cnt)
            def _():
                for s, t, m in copies(bi, d, ph):
                    pltpu.async_copy(s, t, m)

        def compute_idx(krows, slot_b, slot_y, slot_x, slot_i):
            for j in range(K // LANES):
                sl = pl.ds(j * LANES, LANES)
                if j * LANES < krows:
                    if pair:
                        yv = slot_y[sl]
                        yh = lax.bitwise_and(yv, (h // 2) - 1)
                        half = lax.shift_right_logical(
                            yv, (h // 2).bit_length() - 1)
                        r2 = slot_b[sl] * (hw // 2) + yh * w + slot_x[sl]
                        r2 = jnp.minimum(r2, RA - 1)
                        slot_i[0, sl] = jnp.where(half == 0, r2, -1)
                        slot_i[1, sl] = jnp.where(half == 1, r2, -1)
                    else:
                        flat = slot_b[sl] * hw + slot_y[sl] * w + slot_x[sl]
                        flat = jnp.minimum(flat, R - 1)
                        for g in range(gpp):
                            slot_i[g, sl] = flat + g * R
                else:
                    for g in range(GP):
                        slot_i[g, sl] = jnp.full((LANES,), -1, jnp.int32)

        def scatter(d, krows):
            if not pre:
                compute_idx(krows, bbuf.at[d], ybuf.at[d], xbuf.at[d],
                            idxbuf.at[d])
            if pair:
                def shuf(i, _):
                    for j in range(c // LANES):
                        v = stage[d, i, pl.ds(j * LANES, LANES)]
                        fbufL[i, pl.ds(j * LANES, LANES)] = v
                        fbufR[i, pl.ds(c + j * LANES, LANES)] = v
                    return 0
                lax.fori_loop(0, krows, shuf, 0)
                pltpu.async_copy(
                    fbufL, acc.at[plsc.Indices(idxbuf.at[d, 0],
                                               ignored_value=-1)],
                    ssem0, add=True)
                pltpu.async_copy(
                    fbufR, acc.at[plsc.Indices(idxbuf.at[d, 1],
                                               ignored_value=-1)],
                    ssem1, add=True)
                pltpu.make_async_copy(
                    fbufL, acc.at[plsc.Indices(idxbuf.at[d, 0],
                                               ignored_value=-1)],
                    ssem0).wait()
                pltpu.make_async_copy(
                    fbufR, acc.at[plsc.Indices(idxbuf.at[d, 1],
                                               ignored_value=-1)],
                    ssem1).wait()
            else:
                ssems = (ssem0, ssem1)
                for g in range(gpp):
                    pltpu.async_copy(
                        fbuf.at[d, g],
                        acc.at[plsc.Indices(idxbuf.at[d, g],
                                            ignored_value=-1)],
                        ssems[g % 2], add=True)
                for g in range(gpp):
                    pltpu.make_async_copy(
                        fbuf.at[d, g],
                        acc.at[plsc.Indices(idxbuf.at[d, g],
                                            ignored_value=-1)],
                        ssems[g % 2]).wait()

        for ph in range(phases):
            if ph > 0:
                for t in range(rows_per_sub // CZ):
                    r0 = sid * rows_per_sub + t * CZ
                    pltpu.sync_copy(zbuf, acc.at[pl.ds(r0, CZ), :])
            plsc.subcore_barrier()

            def process(bi, d, ph=ph):
                @pl.when(bi < full_cnt)
                def _():
                    for s, t, m in copies(bi, d, ph):
                        pltpu.make_async_copy(s, t, m).wait()
                    issue(bi + 1, d ^ 1, ph)
                    scatter(d, K)

            issue(0, 0, ph)

            def pbody(p, _):
                process(2 * p, 0)
                process(2 * p + 1, 1)
                return 0
            lax.fori_loop(0, (full_cnt + 1) // 2, pbody, 0)

            @pl.when(wid == last_owner)
            def _():
                off = (nblk - 1) * K
                if pair:
                    pltpu.sync_copy(feat.at[pl.ds(off, tail), :],
                                    stage.at[0, pl.ds(0, tail), :])
                else:
                    for g in range(gpp):
                        gc = (ph * gpp + g) * CW
                        pltpu.sync_copy(feat.at[pl.ds(off, tail),
                                                pl.ds(gc, CW)],
                                        fbuf.at[0, g, pl.ds(0, tail), :])
                if pre:
                    assert tail == K
                    pltpu.sync_copy(bidx.at[pl.ds(off, tail)],
                                    idxbuf.at[0, 0])
                else:
                    pltpu.sync_copy(bidx.at[pl.ds(off, tail)],
                                    bbuf.at[0, pl.ds(0, tail)])
                    pltpu.sync_copy(yidx.at[pl.ds(off, tail)],
                                    ybuf.at[0, pl.ds(0, tail)])
                    pltpu.sync_copy(xidx.at[pl.ds(off, tail)],
                                    xbuf.at[0, pl.ds(0, tail)])
                scatter(0, tail)

            plsc.subcore_barrier()

            for t in range(rows_per_sub // CZ):
                r0 = sid * rows_per_sub + t * CZ
                pltpu.sync_copy(acc.at[pl.ds(r0, CZ), :],
                                out.at[cid, pl.ds(ph * RA + r0, CZ), :])

    return sc_kernel


def _make_adapter(hw, cin, cout):
    pair = cin < CW
    G = 1 if pair else cin // CW

    if pair:
        def body(p_ref, w_ref, b_ref, o_ref):
            d = p_ref[0, 0] + p_ref[1, 0]
            we = w_ref[...]
            ot = lax.dot_general(we, d[:, :cin], (((1,), (1,)), ((), ())),
                                 preferred_element_type=jnp.float32)
            ob = lax.dot_general(we, d[:, cin:], (((1,), (1,)), ((), ())),
                                 preferred_element_type=jnp.float32)
            bb = b_ref[0][:, None]
            o_ref[0, :, :hw // 2] = ot + bb
            o_ref[0, :, hw // 2:] = ob + bb

        return pl.pallas_call(
            body,
            grid=(B,),
            in_specs=[
                pl.BlockSpec((NC, 1, hw // 2, CW), lambda b: (0, b, 0, 0)),
                pl.BlockSpec((cout, cin), lambda b: (0, 0)),
                pl.BlockSpec((1, cout), lambda b: (0, 0)),
            ],
            out_specs=pl.BlockSpec((1, cout, hw), lambda b: (b, 0, 0)),
            out_shape=jax.ShapeDtypeStruct((B, cout, hw), jnp.float32),
        )

    def body(p_ref, w_ref, b_ref, o_ref):
        o = None
        for g in range(G):
            d = p_ref[0, g, 0] + p_ref[1, g, 0]
            wg = w_ref[:, g * CW:(g + 1) * CW]
            part = lax.dot_general(wg, d, (((1,), (1,)), ((), ())),
                                   preferred_element_type=jnp.float32)
            o = part if o is None else o + part
        o_ref[0] = o + b_ref[0][:, None]

    return pl.pallas_call(
        body,
        grid=(B,),
        in_specs=[
            pl.BlockSpec((NC, G, 1, hw, CW), lambda b: (0, 0, b, 0, 0)),
            pl.BlockSpec((cout, cin), lambda b: (0, 0)),
            pl.BlockSpec((1, cout), lambda b: (0, 0)),
        ],
        out_specs=pl.BlockSpec((1, cout, hw), lambda b: (b, 0, 0)),
        out_shape=jax.ShapeDtypeStruct((B, cout, hw), jnp.float32),
    )


def _level(feat, bidx, yidx, xidx, W, bvec, h, w, tok):
    n, c = feat.shape
    cout = W.shape[0]
    hw = h * w
    pair = c < CW
    G = 1 if pair else c // CW
    bidx = bidx.astype(jnp.int32)
    yidx = yidx.astype(jnp.int32)
    xidx = xidx.astype(jnp.int32)
    sc = _make_scatter(n, c, h, w)
    parts = sc(feat, bidx, yidx, xidx, tok)
    if pair:
        parts = parts.reshape(NC, B, hw // 2, CW)
    else:
        parts = parts.reshape(NC, G, B, hw, CW)
    mm = _make_adapter(hw, c, cout)
    out = mm(parts, W, bvec.reshape(1, cout))
    return out.reshape(B, cout, h, w), lax.slice(parts.reshape(-1), (0,), (8,))


def kernel(feat0, bidx0, yidx0, xidx0, feat1, bidx1, yidx1, xidx1,
           feat2, bidx2, yidx2, xidx2, W0, b0, W1, b1, W2, b2, batch_size):
    del batch_size
    tok = jnp.zeros((8,), jnp.float32)
    out1, tok = _level(feat1, bidx1, yidx1, xidx1, W1, b1,
                       *LEVEL_SHAPES[1], tok)
    out2, tok = _level(feat2, bidx2, yidx2, xidx2, W2, b2,
                       *LEVEL_SHAPES[2], tok)
    out0, _ = _level(feat0, bidx0, yidx0, xidx0, W0, b0,
                     *LEVEL_SHAPES[0], tok)
    return (out0, out1, out2)


# speedup vs baseline: 3.6015x; 1.0018x over previous
import functools

import jax
import jax.numpy as jnp
from jax import lax
from jax.experimental import pallas as pl
from jax.experimental.pallas import tpu as pltpu
from jax.experimental.pallas import tpu_sc as plsc

B = 4
LEVEL_SHAPES = [(64, 64), (32, 32), (16, 16)]

NC = 2
NSUB = 16
NW = NC * NSUB
LANES = 16
CW = 128


def _zero_buf(buf, nrows, ncols):
    def zrow(i, _):
        for j in range(ncols // LANES):
            buf[i, pl.ds(j * LANES, LANES)] = jnp.zeros((LANES,), jnp.float32)
        return 0
    lax.fori_loop(0, nrows, zrow, 0)


def _make_scatter(n, c, h, w):
    hw = h * w
    R = B * hw
    pair = c < CW
    G = 1 if pair else c // CW
    assert (c == CW // 2) if pair else (c % CW == 0)
    phases = 2 if G > 2 else 1
    gpp = G // phases
    GP = 2 if pair else gpp
    RA = R // 2 if pair else gpp * R
    K = 128 if c == CW else 64
    nblk = -(-n // K)
    tail = n - (nblk - 1) * K
    assert tail % LANES == 0
    base, ext = divmod(nblk, NW)
    last_owner = NW - 1 if base > 0 else ext - 1
    rows_per_sub = RA // NSUB
    assert RA % NSUB == 0
    CZ = min(rows_per_sub, K)

    mesh = plsc.VectorSubcoreMesh(core_axis_name="c", subcore_axis_name="s",
                                  num_cores=NC, num_subcores=NSUB)

    scratch = {
        "acc": pltpu.VMEM_SHARED((RA, CW), jnp.float32),
        "idxbuf": pltpu.VMEM((2, GP, K), jnp.int32),
        "sem0": pltpu.SemaphoreType.DMA,
        "sem1": pltpu.SemaphoreType.DMA,
        "ssem0": pltpu.SemaphoreType.DMA,
        "ssem1": pltpu.SemaphoreType.DMA,
    }
    scratch["bbuf"] = pltpu.VMEM((2, K), jnp.int32)
    scratch["ybuf"] = pltpu.VMEM((2, K), jnp.int32)
    scratch["xbuf"] = pltpu.VMEM((2, K), jnp.int32)
    if pair:
        scratch["stage"] = pltpu.VMEM((2, K, c), jnp.float32)
        scratch["fbufL"] = pltpu.VMEM((K, CW), jnp.float32)
        scratch["fbufR"] = pltpu.VMEM((K, CW), jnp.float32)
    else:
        scratch["fbuf"] = pltpu.VMEM((2, gpp, K, CW), jnp.float32)
        scratch["zbuf"] = pltpu.VMEM((CZ, CW), jnp.float32)

    @functools.partial(
        pl.kernel,
        out_type=jax.ShapeDtypeStruct((NC, phases * RA, CW), jnp.float32),
        mesh=mesh,
        scratch_types=scratch,
    )
    def sc_kernel(feat, bidx, yidx, xidx, tok, out, *, acc, idxbuf,
                  sem0, sem1, ssem0, ssem1, bbuf=None, ybuf=None,
                  xbuf=None, zbuf=None, stage=None, fbufL=None,
                  fbufR=None, fbuf=None):
        del tok
        cid = lax.axis_index("c")
        sid = lax.axis_index("s")
        wid = cid * NSUB + sid
        sems = (sem0, sem1)

        if pair:
            _zero_buf(fbufL, K, CW)
            _zero_buf(fbufR, K, CW)
            zsrc = fbufL
        else:
            _zero_buf(zbuf, CZ, CW)
            zsrc = zbuf
        for t in range(rows_per_sub // CZ):
            r0 = sid * rows_per_sub + t * CZ
            pltpu.sync_copy(zsrc.at[pl.ds(0, CZ), :],
                            acc.at[pl.ds(r0, CZ), :])

        my_base = wid * base + jnp.minimum(wid, ext)
        my_cnt = base + jnp.where(wid < ext, 1, 0)
        full_cnt = my_cnt - jnp.where(wid == last_owner, 1, 0)

        def copies(bi, d, ph):
            off = (my_base + bi) * K
            sem = sems[d]
            cps = []
            if pair:
                cps.append((feat.at[pl.ds(off, K), :], stage.at[d], sem))
            else:
                for g in range(gpp):
                    gc = (ph * gpp + g) * CW
                    cps.append((feat.at[pl.ds(off, K), pl.ds(gc, CW)],
                                fbuf.at[d, g], sem))
            cps.append((bidx.at[pl.ds(off, K)], bbuf.at[d], sem))
            cps.append((yidx.at[pl.ds(off, K)], ybuf.at[d], sem))
            cps.append((xidx.at[pl.ds(off, K)], xbuf.at[d], sem))
            return cps

        def issue(bi, d, ph):
            @pl.when(bi < full_cnt)
            def _():
                for s, t, m in copies(bi, d, ph):
                    pltpu.async_copy(s, t, m)

        def compute_idx(krows, slot_b, slot_y, slot_x, slot_i):
            for j in range(K // LANES):
                sl = pl.ds(j * LANES, LANES)
                if j * LANES < krows:
                    if pair:
                        yv = slot_y[sl]
                        yh = lax.bitwise_and(yv, (h // 2) - 1)
                        half = lax.shift_right_logical(
                            yv, (h // 2).bit_length() - 1)
                        r2 = slot_b[sl] * (hw // 2) + yh * w + slot_x[sl]
                        r2 = jnp.minimum(r2, RA - 1)
                        slot_i[0, sl] = jnp.where(half == 0, r2, -1)
                        slot_i[1, sl] = jnp.where(half == 1, r2, -1)
                    else:
                        flat = slot_b[sl] * hw + slot_y[sl] * w + slot_x[sl]
                        flat = jnp.minimum(flat, R - 1)
                        for g in range(gpp):
                            slot_i[g, sl] = flat + g * R
                else:
                    for g in range(GP):
                        slot_i[g, sl] = jnp.full((LANES,), -1, jnp.int32)

        def scatter(d, krows):
            compute_idx(krows, bbuf.at[d], ybuf.at[d], xbuf.at[d],
                        idxbuf.at[d])
            if pair:
                def shuf(i, _):
                    for j in range(c // LANES):
                        v = stage[d, i, pl.ds(j * LANES, LANES)]
                        fbufL[i, pl.ds(j * LANES, LANES)] = v
                        fbufR[i, pl.ds(c + j * LANES, LANES)] = v
                    return 0
                lax.fori_loop(0, krows, shuf, 0)
                pltpu.async_copy(
                    fbufL, acc.at[plsc.Indices(idxbuf.at[d, 0],
                                               ignored_value=-1)],
                    ssem0, add=True)
                pltpu.async_copy(
                    fbufR, acc.at[plsc.Indices(idxbuf.at[d, 1],
                                               ignored_value=-1)],
                    ssem1, add=True)
                pltpu.make_async_copy(
                    fbufL, acc.at[plsc.Indices(idxbuf.at[d, 0],
                                               ignored_value=-1)],
                    ssem0).wait()
                pltpu.make_async_copy(
                    fbufR, acc.at[plsc.Indices(idxbuf.at[d, 1],
                                               ignored_value=-1)],
                    ssem1).wait()
            else:
                ssems = (ssem0, ssem1)
                for g in range(gpp):
                    pltpu.async_copy(
                        fbuf.at[d, g],
                        acc.at[plsc.Indices(idxbuf.at[d, g],
                                            ignored_value=-1)],
                        ssems[g % 2], add=True)
                for g in range(gpp):
                    pltpu.make_async_copy(
                        fbuf.at[d, g],
                        acc.at[plsc.Indices(idxbuf.at[d, g],
                                            ignored_value=-1)],
                        ssems[g % 2]).wait()

        for ph in range(phases):
            if ph > 0:
                for t in range(rows_per_sub // CZ):
                    r0 = sid * rows_per_sub + t * CZ
                    pltpu.sync_copy(zbuf, acc.at[pl.ds(r0, CZ), :])
            plsc.subcore_barrier()

            def process(bi, d, ph=ph):
                @pl.when(bi < full_cnt)
                def _():
                    for s, t, m in copies(bi, d, ph):
                        pltpu.make_async_copy(s, t, m).wait()
                    issue(bi + 1, d ^ 1, ph)
                    scatter(d, K)

            issue(0, 0, ph)

            def pbody(p, _):
                process(2 * p, 0)
                process(2 * p + 1, 1)
                return 0
            lax.fori_loop(0, (full_cnt + 1) // 2, pbody, 0)

            @pl.when(wid == last_owner)
            def _():
                off = (nblk - 1) * K
                if pair:
                    pltpu.sync_copy(feat.at[pl.ds(off, tail), :],
                                    stage.at[0, pl.ds(0, tail), :])
                else:
                    for g in range(gpp):
                        gc = (ph * gpp + g) * CW
                        pltpu.sync_copy(feat.at[pl.ds(off, tail),
                                                pl.ds(gc, CW)],
                                        fbuf.at[0, g, pl.ds(0, tail), :])
                pltpu.sync_copy(bidx.at[pl.ds(off, tail)],
                                bbuf.at[0, pl.ds(0, tail)])
                pltpu.sync_copy(yidx.at[pl.ds(off, tail)],
                                ybuf.at[0, pl.ds(0, tail)])
                pltpu.sync_copy(xidx.at[pl.ds(off, tail)],
                                xbuf.at[0, pl.ds(0, tail)])
                scatter(0, tail)

            plsc.subcore_barrier()

            for t in range(rows_per_sub // CZ):
                r0 = sid * rows_per_sub + t * CZ
                pltpu.sync_copy(acc.at[pl.ds(r0, CZ), :],
                                out.at[cid, pl.ds(ph * RA + r0, CZ), :])

    return sc_kernel


def _make_adapter(hw, cin, cout):
    pair = cin < CW
    G = 1 if pair else cin // CW

    if pair:
        def body(p_ref, w_ref, b_ref, o_ref):
            d = p_ref[0, 0] + p_ref[1, 0]
            we = w_ref[...]
            ot = lax.dot_general(we, d[:, :cin], (((1,), (1,)), ((), ())),
                                 preferred_element_type=jnp.float32)
            ob = lax.dot_general(we, d[:, cin:], (((1,), (1,)), ((), ())),
                                 preferred_element_type=jnp.float32)
            bb = b_ref[0][:, None]
            o_ref[0, :, :hw // 2] = ot + bb
            o_ref[0, :, hw // 2:] = ob + bb

        return pl.pallas_call(
            body,
            grid=(B,),
            in_specs=[
                pl.BlockSpec((NC, 1, hw // 2, CW), lambda b: (0, b, 0, 0)),
                pl.BlockSpec((cout, cin), lambda b: (0, 0)),
                pl.BlockSpec((1, cout), lambda b: (0, 0)),
            ],
            out_specs=pl.BlockSpec((1, cout, hw), lambda b: (b, 0, 0)),
            out_shape=jax.ShapeDtypeStruct((B, cout, hw), jnp.float32),
        )

    def body(p_ref, w_ref, b_ref, o_ref):
        o = None
        for g in range(G):
            d = p_ref[0, g, 0] + p_ref[1, g, 0]
            wg = w_ref[:, g * CW:(g + 1) * CW]
            part = lax.dot_general(wg, d, (((1,), (1,)), ((), ())),
                                   preferred_element_type=jnp.float32)
            o = part if o is None else o + part
        o_ref[0] = o + b_ref[0][:, None]

    return pl.pallas_call(
        body,
        grid=(B,),
        in_specs=[
            pl.BlockSpec((NC, G, 1, hw, CW), lambda b: (0, 0, b, 0, 0)),
            pl.BlockSpec((cout, cin), lambda b: (0, 0)),
            pl.BlockSpec((1, cout), lambda b: (0, 0)),
        ],
        out_specs=pl.BlockSpec((1, cout, hw), lambda b: (b, 0, 0)),
        out_shape=jax.ShapeDtypeStruct((B, cout, hw), jnp.float32),
    )


def _level(feat, bidx, yidx, xidx, W, bvec, h, w, tok):
    n, c = feat.shape
    cout = W.shape[0]
    hw = h * w
    pair = c < CW
    G = 1 if pair else c // CW
    bidx = bidx.astype(jnp.int32)
    yidx = yidx.astype(jnp.int32)
    xidx = xidx.astype(jnp.int32)
    sc = _make_scatter(n, c, h, w)
    parts = sc(feat, bidx, yidx, xidx, tok)
    if pair:
        parts = parts.reshape(NC, B, hw // 2, CW)
    else:
        parts = parts.reshape(NC, G, B, hw, CW)
    mm = _make_adapter(hw, c, cout)
    out = mm(parts, W, bvec.reshape(1, cout))
    return out.reshape(B, cout, h, w), lax.slice(parts.reshape(-1), (0,), (8,))


def kernel(feat0, bidx0, yidx0, xidx0, feat1, bidx1, yidx1, xidx1,
           feat2, bidx2, yidx2, xidx2, W0, b0, W1, b1, W2, b2, batch_size):
    del batch_size
    tok = jnp.zeros((8,), jnp.float32)
    out1, tok = _level(feat1, bidx1, yidx1, xidx1, W1, b1,
                       *LEVEL_SHAPES[1], tok)
    out2, tok = _level(feat2, bidx2, yidx2, xidx2, W2, b2,
                       *LEVEL_SHAPES[2], tok)
    out0, _ = _level(feat0, bidx0, yidx0, xidx0, W0, b0,
                     *LEVEL_SHAPES[0], tok)
    return (out0, out1, out2)
